# baseline (device time: 88435 ns/iter reference)
import functools

import jax
import jax.numpy as jnp
from jax import lax
from jax.experimental import pallas as pl
from jax.experimental.pallas import tpu as pltpu

N_DEV = 4
N_TOK = 2048
D = 1024
H = 1024
E_LOC = 8
CHUNK = N_TOK // N_DEV


def kernel(x, router_W, route_idx, expert_W):
    x_bf = x.astype(jnp.bfloat16)
    ew_bf = expert_W.astype(jnp.bfloat16).reshape(E_LOC * D, H)

    def body(x_ref, xbf_ref, rw_ref, idx_ref, ew_ref, out_ref,
             acc_ref, gn_ref, lhs_ref, sendb_ref, recvb_ref,
             send_sems, recv_sems):
        my = lax.axis_index("i")
        right = lax.rem(my + 1, N_DEV)
        left = lax.rem(my + 3, N_DEV)

        bar = pltpu.get_barrier_semaphore()
        for nbr in (left, right):
            pl.semaphore_signal(bar, inc=1, device_id=(nbr,),
                                device_id_type=pl.DeviceIdType.MESH)
        pl.semaphore_wait(bar, 2)

        scores = jnp.dot(x_ref[:, :], rw_ref[:, :],
                         preferred_element_type=jnp.float32)
        smax = jnp.max(scores, axis=1, keepdims=True)
        p = jnp.exp(scores - smax)
        e0 = idx_ref[:, 0:1]
        e1 = idx_ref[:, 1:2]
        iota = lax.broadcasted_iota(jnp.int32, (N_TOK, 32), 1)
        g0 = jnp.sum(jnp.where(iota == e0, p, 0.0), axis=1, keepdims=True)
        g1 = jnp.sum(jnp.where(iota == e1, p, 0.0), axis=1, keepdims=True)
        gs = g0 + g1
        gn_ref[:, 0:1] = g0 / gs
        gn_ref[:, 1:2] = g1 / gs

        def partial_into_acc(c):
            r0 = c * CHUNK
            rows = pl.ds(r0, CHUNK)
            xb = xbf_ref[rows, :]
            e0c = idx_ref[rows, 0:1]
            e1c = idx_ref[rows, 1:2]
            g0c = gn_ref[rows, 0:1]
            g1c = gn_ref[rows, 1:2]
            for j in range(E_LOC):
                ej = my * E_LOC + j
                w = (jnp.where(e0c == ej, g0c, 0.0)
                     + jnp.where(e1c == ej, g1c, 0.0))
                lhs_ref[:, j * D:(j + 1) * D] = xb * w.astype(jnp.bfloat16)
            acc_ref[:, :] = jnp.dot(lhs_ref[:, :], ew_ref[:, :],
                                    preferred_element_type=jnp.float32)

        partial_into_acc(lax.rem(my + 3, N_DEV))
        for s in range(N_DEV - 1):
            sendb_ref[s] = acc_ref[:, :].astype(jnp.bfloat16)
            rdma = pltpu.make_async_remote_copy(
                src_ref=sendb_ref.at[s],
                dst_ref=recvb_ref.at[s],
                send_sem=send_sems.at[s],
                recv_sem=recv_sems.at[s],
                device_id=(right,),
                device_id_type=pl.DeviceIdType.MESH,
            )
            rdma.start()
            partial_into_acc(lax.rem(my + 2 - s + N_DEV, N_DEV))
            rdma.wait()
            acc_ref[:, :] = acc_ref[:, :] + recvb_ref[s].astype(jnp.float32)

        out_ref[:, :] = acc_ref[:, :]

        @functools.partial(pl.run_scoped, sem2=pltpu.SemaphoreType.REGULAR)
        def _(sem2):
            for nbr in (left, right):
                pl.semaphore_signal(sem2, inc=1, device_id=(nbr,),
                                    device_id_type=pl.DeviceIdType.MESH)
            pl.semaphore_wait(sem2, 2)

    return pl.pallas_call(
        body,
        out_shape=jax.ShapeDtypeStruct((CHUNK, H), jnp.float32),
        in_specs=[pl.BlockSpec(memory_space=pltpu.VMEM)] * 5,
        out_specs=pl.BlockSpec(memory_space=pltpu.VMEM),
        scratch_shapes=[
            pltpu.VMEM((CHUNK, H), jnp.float32),
            pltpu.VMEM((N_TOK, 2), jnp.float32),
            pltpu.VMEM((CHUNK, E_LOC * D), jnp.bfloat16),
            pltpu.VMEM((N_DEV - 1, CHUNK, H), jnp.bfloat16),
            pltpu.VMEM((N_DEV - 1, CHUNK, H), jnp.bfloat16),
            pltpu.SemaphoreType.DMA((N_DEV - 1,)),
            pltpu.SemaphoreType.DMA((N_DEV - 1,)),
        ],
        compiler_params=pltpu.CompilerParams(collective_id=0),
    )(x, x_bf, router_W, route_idx, ew_bf)


# device time: 70662 ns/iter; 1.2515x vs baseline; 1.2515x over previous
import functools

import jax
import jax.numpy as jnp
from jax import lax
from jax.experimental import pallas as pl
from jax.experimental.pallas import tpu as pltpu

N_DEV = 4
N_TOK = 2048
D = 1024
H = 1024
E_LOC = 8
CHUNK = N_TOK // N_DEV


def kernel(x, router_W, route_idx, expert_W):

    def body(x_ref, rw_ref, idx_ref, ew_hbm, out_ref,
             ewbf_ref, stage_ref, xbf_ref, acc_ref, gn_ref,
             sendb_ref, recvb_ref, stage_sems, send_sems, recv_sems):
        my = lax.axis_index("i")
        right = lax.rem(my + 1, N_DEV)
        left = lax.rem(my + 3, N_DEV)

        bar = pltpu.get_barrier_semaphore()
        for nbr in (left, right):
            pl.semaphore_signal(bar, inc=1, device_id=(nbr,),
                                device_id_type=pl.DeviceIdType.MESH)
        pl.semaphore_wait(bar, 2)

        def stage_copy(j):
            return pltpu.make_async_copy(
                ew_hbm.at[j], stage_ref.at[j % 2], stage_sems.at[j % 2])

        stage_copy(0).start()

        xbf_ref[:, :] = x_ref[:, :].astype(jnp.bfloat16)
        scores = jnp.dot(x_ref[:, :], rw_ref[:, :],
                         preferred_element_type=jnp.float32)
        smax = jnp.max(scores, axis=1, keepdims=True)
        p = jnp.exp(scores - smax)
        e0 = idx_ref[:, 0:1]
        e1 = idx_ref[:, 1:2]
        iota = lax.broadcasted_iota(jnp.int32, (N_TOK, 32), 1)
        g0 = jnp.sum(jnp.where(iota == e0, p, 0.0), axis=1, keepdims=True)
        g1 = jnp.sum(jnp.where(iota == e1, p, 0.0), axis=1, keepdims=True)
        gs = g0 + g1
        gn_ref[:, 0:1] = g0 / gs
        gn_ref[:, 1:2] = g1 / gs

        def expert_contrib(c, j, accumulate):
            rows = pl.ds(c * CHUNK, CHUNK)
            ej = my * E_LOC + j
            w = (jnp.where(idx_ref[rows, 0:1] == ej, gn_ref[rows, 0:1], 0.0)
                 + jnp.where(idx_ref[rows, 1:2] == ej, gn_ref[rows, 1:2], 0.0))
            y = jnp.dot(xbf_ref[rows, :], ewbf_ref[j],
                        preferred_element_type=jnp.float32)
            if accumulate:
                acc_ref[:, :] = acc_ref[:, :] + y * w
            else:
                acc_ref[:, :] = y * w

        c_first = lax.rem(my + 3, N_DEV)
        for j in range(E_LOC):
            if j + 1 < E_LOC:
                stage_copy(j + 1).start()
            stage_copy(j).wait()
            ewbf_ref[j] = stage_ref[j % 2].astype(jnp.bfloat16)
            expert_contrib(c_first, j, accumulate=(j > 0))

        for s in range(N_DEV - 1):
            sendb_ref[s] = acc_ref[:, :].astype(jnp.bfloat16)
            rdma = pltpu.make_async_remote_copy(
                src_ref=sendb_ref.at[s],
                dst_ref=recvb_ref.at[s],
                send_sem=send_sems.at[s],
                recv_sem=recv_sems.at[s],
                device_id=(right,),
                device_id_type=pl.DeviceIdType.MESH,
            )
            rdma.start()
            c_next = lax.rem(my + 2 - s + N_DEV, N_DEV)
            for j in range(E_LOC):
                expert_contrib(c_next, j, accumulate=(j > 0))
            rdma.wait()
            acc_ref[:, :] = acc_ref[:, :] + recvb_ref[s].astype(jnp.float32)

        out_ref[:, :] = acc_ref[:, :]

        @functools.partial(pl.run_scoped, sem2=pltpu.SemaphoreType.REGULAR)
        def _(sem2):
            for nbr in (left, right):
                pl.semaphore_signal(sem2, inc=1, device_id=(nbr,),
                                    device_id_type=pl.DeviceIdType.MESH)
            pl.semaphore_wait(sem2, 2)

    return pl.pallas_call(
        body,
        out_shape=jax.ShapeDtypeStruct((CHUNK, H), jnp.float32),
        in_specs=[
            pl.BlockSpec(memory_space=pltpu.VMEM),
            pl.BlockSpec(memory_space=pltpu.VMEM),
            pl.BlockSpec(memory_space=pltpu.VMEM),
            pl.BlockSpec(memory_space=pl.ANY),
        ],
        out_specs=pl.BlockSpec(memory_space=pltpu.VMEM),
        scratch_shapes=[
            pltpu.VMEM((E_LOC, D, H), jnp.bfloat16),
            pltpu.VMEM((2, D, H), jnp.float32),
            pltpu.VMEM((N_TOK, D), jnp.bfloat16),
            pltpu.VMEM((CHUNK, H), jnp.float32),
            pltpu.VMEM((N_TOK, 2), jnp.float32),
            pltpu.VMEM((N_DEV - 1, CHUNK, H), jnp.bfloat16),
            pltpu.VMEM((N_DEV - 1, CHUNK, H), jnp.bfloat16),
            pltpu.SemaphoreType.DMA((2,)),
            pltpu.SemaphoreType.DMA((N_DEV - 1,)),
            pltpu.SemaphoreType.DMA((N_DEV - 1,)),
        ],
        compiler_params=pltpu.CompilerParams(
            collective_id=0,
            vmem_limit_bytes=50 * 1024 * 1024,
        ),
    )(x, router_W, route_idx, expert_W)


# device time: 58197 ns/iter; 1.5196x vs baseline; 1.2142x over previous
import functools

import jax
import jax.numpy as jnp
from jax import lax
from jax.experimental import pallas as pl
from jax.experimental.pallas import tpu as pltpu

N_DEV = 4
N_TOK = 2048
D = 1024
H = 1024
E_LOC = 8
CHUNK = N_TOK // N_DEV
HALF = CHUNK // 2


def kernel(x, router_W, route_idx, expert_W):

    def body(x_ref, rw_ref, idx_ref, ew_hbm, out_ref,
             ewbf_ref, stage_ref, xbf_ref, accA_ref, accB_ref, gn_ref,
             sendA_ref, recvA_ref, sendB_ref, recvB_ref,
             stage_sems, semsA_s, semsA_r, semsB_s, semsB_r):
        my = lax.axis_index("i")
        right = lax.rem(my + 1, N_DEV)
        left = lax.rem(my + 3, N_DEV)

        def stage_copy(j):
            return pltpu.make_async_copy(
                ew_hbm.at[j], stage_ref.at[j % 2], stage_sems.at[j % 2])

        stage_copy(0).start()
        stage_copy(1).start()

        bar = pltpu.get_barrier_semaphore()
        for nbr in (left, right):
            pl.semaphore_signal(bar, inc=1, device_id=(nbr,),
                                device_id_type=pl.DeviceIdType.MESH)
        pl.semaphore_wait(bar, 2)

        xbf_ref[:, :] = x_ref[:, :].astype(jnp.bfloat16)
        scores = jnp.dot(x_ref[:, :], rw_ref[:, :],
                         preferred_element_type=jnp.float32)
        smax = jnp.max(scores, axis=1, keepdims=True)
        p = jnp.exp(scores - smax)
        e0 = idx_ref[:, 0:1]
        e1 = idx_ref[:, 1:2]
        iota = lax.broadcasted_iota(jnp.int32, (N_TOK, 32), 1)
        g0 = jnp.sum(jnp.where(iota == e0, p, 0.0), axis=1, keepdims=True)
        g1 = jnp.sum(jnp.where(iota == e1, p, 0.0), axis=1, keepdims=True)
        gs = g0 + g1
        gn_ref[:, 0:1] = g0 / gs
        gn_ref[:, 1:2] = g1 / gs

        def expert_contrib(c, off, j, acc_ref, accumulate):
            rows = pl.ds(c * CHUNK + off, HALF)
            ej = my * E_LOC + j
            w = (jnp.where(idx_ref[rows, 0:1] == ej, gn_ref[rows, 0:1], 0.0)
                 + jnp.where(idx_ref[rows, 1:2] == ej, gn_ref[rows, 1:2], 0.0))
            y = jnp.dot(xbf_ref[rows, :], ewbf_ref[j],
                        preferred_element_type=jnp.float32)
            if accumulate:
                acc_ref[:, :] = acc_ref[:, :] + y * w
            else:
                acc_ref[:, :] = y * w

        def compute_pair(cA, cB, j):
            expert_contrib(cA, 0, j, accA_ref, accumulate=(j > 0))
            expert_contrib(cB, HALF, j, accB_ref, accumulate=(j > 0))

        for j in range(E_LOC):
            stage_copy(j).wait()
            ewbf_ref[j] = stage_ref[j % 2].astype(jnp.bfloat16)
            if j + 2 < E_LOC:
                stage_copy(j + 2).start()
            compute_pair(lax.rem(my + 3, N_DEV), lax.rem(my + 1, N_DEV), j)

        for s in range(N_DEV - 1):
            sendA_ref[s] = accA_ref[:, :].astype(jnp.bfloat16)
            sendB_ref[s] = accB_ref[:, :].astype(jnp.bfloat16)
            rdmaA = pltpu.make_async_remote_copy(
                src_ref=sendA_ref.at[s], dst_ref=recvA_ref.at[s],
                send_sem=semsA_s.at[s], recv_sem=semsA_r.at[s],
                device_id=(right,), device_id_type=pl.DeviceIdType.MESH)
            rdmaB = pltpu.make_async_remote_copy(
                src_ref=sendB_ref.at[s], dst_ref=recvB_ref.at[s],
                send_sem=semsB_s.at[s], recv_sem=semsB_r.at[s],
                device_id=(left,), device_id_type=pl.DeviceIdType.MESH)
            rdmaA.start()
            rdmaB.start()
            for j in range(E_LOC):
                compute_pair(lax.rem(my + 2 - s + N_DEV, N_DEV),
                             lax.rem(my + 2 + s, N_DEV), j)
            rdmaA.wait()
            accA_ref[:, :] = accA_ref[:, :] + recvA_ref[s].astype(jnp.float32)
            rdmaB.wait()
            accB_ref[:, :] = accB_ref[:, :] + recvB_ref[s].astype(jnp.float32)

        out_ref[0:HALF, :] = accA_ref[:, :]
        out_ref[HALF:CHUNK, :] = accB_ref[:, :]

        @functools.partial(pl.run_scoped, sem2=pltpu.SemaphoreType.REGULAR)
        def _(sem2):
            for nbr in (left, right):
                pl.semaphore_signal(sem2, inc=1, device_id=(nbr,),
                                    device_id_type=pl.DeviceIdType.MESH)
            pl.semaphore_wait(sem2, 2)

    return pl.pallas_call(
        body,
        out_shape=jax.ShapeDtypeStruct((CHUNK, H), jnp.float32),
        in_specs=[
            pl.BlockSpec(memory_space=pltpu.VMEM),
            pl.BlockSpec(memory_space=pltpu.VMEM),
            pl.BlockSpec(memory_space=pltpu.VMEM),
            pl.BlockSpec(memory_space=pl.ANY),
        ],
        out_specs=pl.BlockSpec(memory_space=pltpu.VMEM),
        scratch_shapes=[
            pltpu.VMEM((E_LOC, D, H), jnp.bfloat16),
            pltpu.VMEM((2, D, H), jnp.float32),
            pltpu.VMEM((N_TOK, D), jnp.bfloat16),
            pltpu.VMEM((HALF, H), jnp.float32),
            pltpu.VMEM((HALF, H), jnp.float32),
            pltpu.VMEM((N_TOK, 2), jnp.float32),
            pltpu.VMEM((N_DEV - 1, HALF, H), jnp.bfloat16),
            pltpu.VMEM((N_DEV - 1, HALF, H), jnp.bfloat16),
            pltpu.VMEM((N_DEV - 1, HALF, H), jnp.bfloat16),
            pltpu.VMEM((N_DEV - 1, HALF, H), jnp.bfloat16),
            pltpu.SemaphoreType.DMA((2,)),
            pltpu.SemaphoreType.DMA((N_DEV - 1,)),
            pltpu.SemaphoreType.DMA((N_DEV - 1,)),
            pltpu.SemaphoreType.DMA((N_DEV - 1,)),
            pltpu.SemaphoreType.DMA((N_DEV - 1,)),
        ],
        compiler_params=pltpu.CompilerParams(
            collective_id=0,
            vmem_limit_bytes=50 * 1024 * 1024,
        ),
    )(x, router_W, route_idx, expert_W)
